# Initial kernel scaffold; baseline (speedup 1.0000x reference)
import jax
import jax.numpy as jnp
from jax.experimental import pallas as pl


def kernel(x, edge_index, W_self0, W_neigh0, b0, W_self1, W_neigh1, b1, W_self2, W_neigh2, b2):
    raise NotImplementedError("write your pallas kernel here")


# speedup vs baseline: 6.4037x; 6.4037x over previous
import functools

import jax
import jax.numpy as jnp
from jax import lax
from jax.experimental import pallas as pl
from jax.experimental.pallas import tpu as pltpu
from jax.experimental.pallas import tpu_sc as plsc

_NC = 2
_NS = 16
_CH = 128

_SELU_ALPHA = 1.6732632423543772
_SELU_LAM = 1.0507009873554805


def _sc_segsum(with_deg, n_pad, d, e):
    nw = _NC * _NS
    ew = e // nw
    fc = ew // _CH
    rem = ew % _CH
    rps = n_pad // _NS

    out_type = [jax.ShapeDtypeStruct((_NC, n_pad, d), jnp.float32)]
    scratch = [
        pltpu.VMEM((_CH,), jnp.int32),
        pltpu.VMEM((_CH,), jnp.int32),
        pltpu.VMEM((_CH, d), jnp.float32),
        pltpu.VMEM_SHARED((n_pad, d), jnp.float32),
        pltpu.SemaphoreType.DMA,
    ]
    if with_deg:
        out_type.append(jax.ShapeDtypeStruct((_NC, n_pad), jnp.float32))
        scratch += [
            pltpu.VMEM((_CH,), jnp.float32),
            pltpu.VMEM_SHARED((n_pad,), jnp.float32),
        ]
    if rem:
        scratch += [
            pltpu.VMEM((rem,), jnp.int32),
            pltpu.VMEM((rem,), jnp.int32),
            pltpu.VMEM((rem, d), jnp.float32),
        ]
        if with_deg:
            scratch.append(pltpu.VMEM((rem,), jnp.float32))

    mesh = plsc.VectorSubcoreMesh(core_axis_name="c", subcore_axis_name="s")

    def body(*refs):
        if with_deg:
            h_hbm, src_hbm, dst_hbm, z2_hbm, z1_hbm = refs[:5]
            refs = refs[5:]
            agg_out, deg_out = refs[:2]
            refs = refs[2:]
        else:
            h_hbm, src_hbm, dst_hbm, z2_hbm = refs[:4]
            refs = refs[4:]
            (agg_out,) = refs[:1]
            refs = refs[1:]
        src_v, dst_v, rows_v, acc, sem = refs[:5]
        refs = refs[5:]
        if with_deg:
            ones_v, dacc = refs[:2]
            refs = refs[2:]
        if rem:
            srcr_v, dstr_v, rowsr_v = refs[:3]
            refs = refs[3:]
            if with_deg:
                (onesr_v,) = refs

        cid = lax.axis_index("c")
        sid = lax.axis_index("s")
        base = pl.multiple_of(sid * rps, 8)

        pltpu.sync_copy(z2_hbm.at[pl.ds(base, rps)], acc.at[pl.ds(base, rps)])
        if with_deg:
            pltpu.sync_copy(z1_hbm.at[pl.ds(base, rps)], dacc.at[pl.ds(base, rps)])
            for i in range(_CH // 16):
                ones_v[pl.ds(i * 16, 16)] = jnp.ones((16,), jnp.float32)
            if rem:
                for i in range(rem // 16):
                    onesr_v[pl.ds(i * 16, 16)] = jnp.ones((16,), jnp.float32)
        plsc.subcore_barrier()

        wid = sid * _NC + cid
        ebase = wid * ew

        def chunk(j, carry):
            off = pl.multiple_of(ebase + j * _CH, 8)
            pltpu.sync_copy(src_hbm.at[pl.ds(off, _CH)], src_v)
            pltpu.sync_copy(dst_hbm.at[pl.ds(off, _CH)], dst_v)
            pltpu.async_copy(h_hbm.at[src_v], rows_v, sem).wait()
            pltpu.sync_copy(rows_v, acc.at[dst_v], add=True)
            if with_deg:
                pltpu.sync_copy(ones_v, dacc.at[dst_v], add=True)
            return carry

        lax.fori_loop(0, fc, chunk, 0)

        if rem:
            off = pl.multiple_of(ebase + fc * _CH, 8)
            pltpu.sync_copy(src_hbm.at[pl.ds(off, rem)], srcr_v)
            pltpu.sync_copy(dst_hbm.at[pl.ds(off, rem)], dstr_v)
            pltpu.async_copy(h_hbm.at[srcr_v], rowsr_v, sem).wait()
            pltpu.sync_copy(rowsr_v, acc.at[dstr_v], add=True)
            if with_deg:
                pltpu.sync_copy(onesr_v, dacc.at[dstr_v], add=True)

        plsc.subcore_barrier()
        pltpu.sync_copy(acc.at[pl.ds(base, rps)],
                        agg_out.at[cid, pl.ds(base, rps)])
        if with_deg:
            pltpu.sync_copy(dacc.at[pl.ds(base, rps)],
                            deg_out.at[cid, pl.ds(base, rps)])

    return pl.kernel(body, out_type=tuple(out_type), mesh=mesh,
                     scratch_types=tuple(scratch))


def _tc_layer(h, parts, deg_parts, w_self, w_neigh, b, act, block_rows):
    n_pad, d = h.shape
    hdim = w_self.shape[1]
    grid = n_pad // block_rows

    def body(h_ref, p_ref, dg_ref, ws_ref, wn_ref, b_ref, o_ref):
        hb = h_ref[...]
        agg = p_ref[0] + p_ref[1]
        deg = dg_ref[0] + dg_ref[1]
        mean = agg * (1.0 / jnp.maximum(deg, 1.0))
        y = (jnp.dot(hb, ws_ref[...], preferred_element_type=jnp.float32)
             + jnp.dot(mean, wn_ref[...], preferred_element_type=jnp.float32)
             + b_ref[...])
        if act == "selu":
            o_ref[...] = jnp.where(
                y > 0.0, _SELU_LAM * y,
                (_SELU_LAM * _SELU_ALPHA) * (jnp.exp(y) - 1.0))
        else:
            m = jnp.max(y, axis=1, keepdims=True)
            ey = jnp.exp(y - m)
            o_ref[...] = ey / jnp.sum(ey, axis=1, keepdims=True)

    return pl.pallas_call(
        body,
        grid=(grid,),
        in_specs=[
            pl.BlockSpec((block_rows, d), lambda i: (i, 0)),
            pl.BlockSpec((_NC, block_rows, d), lambda i: (0, i, 0)),
            pl.BlockSpec((_NC, block_rows, 1), lambda i: (0, i, 0)),
            pl.BlockSpec((d, hdim), lambda i: (0, 0)),
            pl.BlockSpec((d, hdim), lambda i: (0, 0)),
            pl.BlockSpec((1, hdim), lambda i: (0, 0)),
        ],
        out_specs=pl.BlockSpec((block_rows, hdim), lambda i: (i, 0)),
        out_shape=jax.ShapeDtypeStruct((n_pad, hdim), jnp.float32),
    )(h, parts, deg_parts, w_self, w_neigh, b)


def kernel(x, edge_index, W_self0, W_neigh0, b0, W_self1, W_neigh1, b1,
           W_self2, W_neigh2, b2):
    n, d = x.shape
    e = edge_index.shape[1]
    n_pad = -(-n // 1280) * 1280

    src = edge_index[0]
    dst = edge_index[1]
    xp = jnp.zeros((n_pad, d), jnp.float32).at[:n].set(x)
    z2 = jnp.zeros((n_pad, d), jnp.float32)
    z1 = jnp.zeros((n_pad,), jnp.float32)

    seg_deg = _sc_segsum(True, n_pad, d, e)
    seg = _sc_segsum(False, n_pad, d, e)

    agg0, deg = seg_deg(xp, src, dst, z2, z1)
    degr = deg.reshape(_NC, n_pad, 1)
    b0r = b0.reshape(1, -1)
    b1r = b1.reshape(1, -1)
    b2r = b2.reshape(1, -1)

    h1 = _tc_layer(xp, agg0, degr, W_self0, W_neigh0, b0r, "selu", 640)
    (agg1,) = seg(h1, src, dst, z2)
    h2 = _tc_layer(h1, agg1, degr, W_self1, W_neigh1, b1r, "selu", 640)
    (agg2,) = seg(h2, src, dst, z2)
    out = _tc_layer(h2, agg2, degr, W_self2, W_neigh2, b2r, "softmax", 640)
    return out[:n]
